# baseline (speedup 1.0000x reference)
import functools

import jax
import jax.numpy as jnp
from jax import lax
from jax.experimental import pallas as pl
from jax.experimental.pallas import tpu as pltpu
from jax.experimental.pallas import tpu_sc as plsc

B = 128
N = 32768
KCOL = 1000
L = 16
NEG = float("-inf")

SPECIAL = (KCOL // L) * L
SPECIAL_LANE = KCOL - SPECIAL
AFTER = SPECIAL + L

NC = 2
NS = 16
NW = NC * NS
ROWS_PER_W = B // NW

ZN = 16384
UNROLL = 8


def _span_max(ref, start, nelems, accs):
    nv = nelems // L
    nsteps = nv // UNROLL
    nmain = nsteps * UNROLL
    accs = list(accs)
    if nsteps:
        def body(i, a4):
            base = start + i * (L * UNROLL)
            a = list(a4)
            for u in range(UNROLL):
                x = ref[pl.ds(base + u * L, L)]
                a[u % 4] = jnp.maximum(a[u % 4], x)
            return tuple(a)
        accs = list(lax.fori_loop(0, nsteps, body, tuple(accs)))
    for j in range(nmain, nv):
        x = ref[pl.ds(start + j * L, L)]
        accs[j % 4] = jnp.maximum(accs[j % 4], x)
    return tuple(accs)


def _red4(accs):
    a0, a1, a2, a3 = accs
    return jnp.maximum(jnp.maximum(a0, a1), jnp.maximum(a2, a3))


_DNUMS = lax.GatherDimensionNumbers(
    offset_dims=(), collapsed_slice_dims=(0,), start_index_map=(0,))


def _shuffle(x, idx):
    return lax.gather(x, jnp.reshape(idx, (L, 1)), _DNUMS, slice_sizes=(1,),
                      mode=lax.GatherScatterMode.PROMISE_IN_BOUNDS)


def _xlane_max(x, lane):
    for s in (1, 2, 4, 8):
        x = jnp.maximum(x, _shuffle(x, jnp.bitwise_xor(lane, s)))
    return x


def _body(in_hbm, out_hbm, ibuf0, ibuf1, zbuf, fbuf, sem0, sem1, semw):
    wid = lax.axis_index("s") * NC + lax.axis_index("c")
    row0 = wid * ROWS_PER_W
    ibufs = (ibuf0, ibuf1)
    insems = (sem0, sem1)

    zv = jnp.zeros((L,), jnp.float32)

    def zbody(i, carry):
        zbuf[pl.ds(i * L, L)] = zv
        return carry
    lax.fori_loop(0, ZN // L, zbody, 0)

    lane = lax.iota(jnp.int32, L)
    neg = jnp.full((L,), NEG, jnp.float32)

    copies = [None] * ROWS_PER_W
    copies[0] = pltpu.async_copy(in_hbm.at[pl.ds(row0 * N, N)], ibufs[0], insems[0])

    writes = []
    for r in range(ROWS_PER_W):
        row = row0 + r
        if r + 1 < ROWS_PER_W:
            copies[r + 1] = pltpu.async_copy(
                in_hbm.at[pl.ds((row0 + r + 1) * N, N)],
                ibufs[(r + 1) % 2], insems[(r + 1) % 2])
        copies[r].wait()
        ibuf = ibufs[r % 2]

        accs_b = _span_max(ibuf, 0, SPECIAL, (neg, neg, neg, neg))
        accs_a = _span_max(ibuf, AFTER, N - AFTER, (neg, neg, neg, neg))
        xs = ibuf[pl.ds(SPECIAL, L)]
        ab = _xlane_max(
            jnp.maximum(_red4(accs_b), jnp.where(lane < SPECIAL_LANE, xs, neg)),
            lane)
        aa = _xlane_max(
            jnp.maximum(_red4(accs_a), jnp.where(lane > SPECIAL_LANE, xs, neg)),
            lane)
        vv = _shuffle(xs, jnp.full((L,), SPECIAL_LANE, jnp.int32))
        condv = jnp.logical_and(vv > ab, vv >= aa)
        fvec = jnp.where(jnp.logical_and(lane == SPECIAL_LANE, condv),
                         jnp.float32(1.0), jnp.float32(0.0))
        fbuf[r] = fvec

        base = row * N
        writes.append(pltpu.async_copy(
            zbuf.at[pl.ds(0, SPECIAL)], out_hbm.at[pl.ds(base, SPECIAL)], semw))
        writes.append(pltpu.async_copy(
            fbuf.at[r], out_hbm.at[pl.ds(base + SPECIAL, L)], semw))
        writes.append(pltpu.async_copy(
            zbuf.at[pl.ds(0, ZN)], out_hbm.at[pl.ds(base + AFTER, ZN)], semw))
        writes.append(pltpu.async_copy(
            zbuf.at[pl.ds(0, N - AFTER - ZN)],
            out_hbm.at[pl.ds(base + AFTER + ZN, N - AFTER - ZN)], semw))

    for w in writes:
        w.wait()


@functools.partial(jax.jit, donate_argnums=())
def _run(inputs):
    mesh = plsc.VectorSubcoreMesh(core_axis_name="c", subcore_axis_name="s")
    f = pl.kernel(
        _body,
        out_type=jax.ShapeDtypeStruct((B * N,), jnp.float32),
        mesh=mesh,
        scratch_types=[
            pltpu.VMEM((N,), jnp.float32),
            pltpu.VMEM((N,), jnp.float32),
            pltpu.VMEM((ZN,), jnp.float32),
            pltpu.VMEM((ROWS_PER_W, L), jnp.float32),
            pltpu.SemaphoreType.DMA,
            pltpu.SemaphoreType.DMA,
            pltpu.SemaphoreType.DMA,
        ],
    )
    return f(inputs.reshape(B * N)).reshape(B, N)


def kernel(inputs):
    return _run(inputs)


# speedup vs baseline: 2.1516x; 2.1516x over previous
import functools

import jax
import jax.numpy as jnp
from jax import lax
from jax.experimental import pallas as pl
from jax.experimental.pallas import tpu as pltpu
from jax.experimental.pallas import tpu_sc as plsc

B = 128
N = 32768
KCOL = 1000
L = 16
NEG = float("-inf")

NC = 2
NS = 16
G = 8
NG = B // G
HALF = N // 2

CHUNK = 4096
NCH = HALF // CHUNK

FT0 = (KCOL // 128) * 128
SPECIAL = (KCOL // L) * L
SPECIAL_LANE = KCOL - SPECIAL
SPECIAL_VREG = SPECIAL // L

ZW = 1024


_DNUMS = lax.GatherDimensionNumbers(
    offset_dims=(), collapsed_slice_dims=(0,), start_index_map=(0,))


def _shuffle(x, idx):
    return lax.gather(x, jnp.reshape(idx, (L, 1)), _DNUMS, slice_sizes=(1,),
                      mode=lax.GatherScatterMode.PROMISE_IN_BOUNDS)


def _xlane_max(x, lane):
    for s in (1, 2, 4, 8):
        x = jnp.maximum(x, _shuffle(x, jnp.bitwise_xor(lane, s)))
    return x


def _rows_max(ref, lo, hi, accs):
    if hi <= lo:
        return tuple(accs)
    if hi - lo <= 2:
        a = list(accs)
        for j in range(lo, hi):
            for i in range(G):
                a[i] = jnp.maximum(a[i], ref[i, pl.ds(j * L, L)])
        return tuple(a)

    def body(j, a8):
        base = j * L
        a = list(a8)
        for i in range(G):
            a[i] = jnp.maximum(a[i], ref[i, pl.ds(base, L)])
        return tuple(a)
    return lax.fori_loop(lo, hi, body, tuple(accs))


def _zero_fill(ref, w):
    zv = jnp.zeros((L,), jnp.float32)

    def body(j, c):
        for i in range(G):
            ref[i, pl.ds(j * L, L)] = zv
        return c
    lax.fori_loop(0, w // L, body, 0)


def _body(in_hbm, out_hbm, ibuf0, ibuf1, zb, ft, xbuf, shared,
          sem0, sem1, semw, semx):
    c = lax.axis_index("c")
    s = lax.axis_index("s")
    g = c * (NG // NC) + s // 2
    h = s % 2
    r0 = pl.multiple_of(g * G, G)
    c0 = pl.multiple_of(h * HALF, HALF)

    ibufs = (ibuf0, ibuf1)
    insems = (sem0, sem1)
    lane = lax.iota(jnp.int32, L)
    neg = jnp.full((L,), NEG, jnp.float32)
    zv = jnp.zeros((L,), jnp.float32)
    one = jnp.full((L,), 1.0, jnp.float32)
    hvec = jnp.full((L,), h, jnp.int32)

    _zero_fill(zb, ZW)
    _zero_fill(ft, 128)

    copies = [None] * NCH
    copies[0] = pltpu.async_copy(
        in_hbm.at[pl.ds(r0, G), pl.ds(c0, CHUNK)], ibufs[0], insems[0])

    writes = [pltpu.async_copy(
        zb.at[:, pl.ds(0, FT0)],
        out_hbm.at[pl.ds(r0, G), pl.ds(c0, FT0)], semw)]
    for k in range(1, HALF // ZW):
        writes.append(pltpu.async_copy(
            zb, out_hbm.at[pl.ds(r0, G), pl.ds(c0 + k * ZW, ZW)], semw))

    acc_b = [neg] * G
    acc_a = [neg] * G
    xs = [None] * G
    for ch in range(NCH):
        if ch + 1 < NCH:
            copies[ch + 1] = pltpu.async_copy(
                in_hbm.at[pl.ds(r0, G), pl.ds(c0 + (ch + 1) * CHUNK, CHUNK)],
                ibufs[(ch + 1) % 2], insems[(ch + 1) % 2])
        copies[ch].wait()
        ibuf = ibufs[ch % 2]
        if ch == 0:
            acc_b = list(_rows_max(ibuf, 0, SPECIAL_VREG, acc_b))
            acc_a = list(_rows_max(ibuf, SPECIAL_VREG + 1, CHUNK // L, acc_a))
            xs = [ibuf[i, pl.ds(SPECIAL, L)] for i in range(G)]
        else:
            acc_a = list(_rows_max(ibuf, 0, CHUNK // L, acc_a))

    keepf = jnp.maximum(jnp.where(lane > SPECIAL_LANE, one, zv),
                        jnp.where(hvec == 1, one, zv))
    half_one = jnp.full((L,), 0.5, jnp.float32)
    for i in range(G):
        x = xs[i]
        acc_b[i] = jnp.maximum(acc_b[i],
                               jnp.where(lane < SPECIAL_LANE, x, neg))
        acc_a[i] = jnp.maximum(acc_a[i], jnp.where(keepf > half_one, x, neg))

    mb_pack = neg
    ma_pack = neg
    v_pack = neg
    for i in range(G):
        bmax = _xlane_max(acc_b[i], lane)
        amax = _xlane_max(acc_a[i], lane)
        vbc = _shuffle(xs[i], jnp.full((L,), SPECIAL_LANE, jnp.int32))
        mb_pack = jnp.maximum(mb_pack, jnp.where(lane == i, bmax, neg))
        ma_pack = jnp.maximum(ma_pack, jnp.where(lane == i, amax, neg))
        v_pack = jnp.maximum(v_pack, jnp.where(lane == i, vbc, neg))

    xbuf[...] = ma_pack
    pltpu.async_copy(xbuf, shared.at[s], semx).wait()
    plsc.subcore_barrier()
    pltpu.async_copy(shared.at[jnp.bitwise_xor(s, 1)], xbuf, semx).wait()
    pa = xbuf[...]

    condf = (jnp.where(v_pack > mb_pack, one, zv)
             * jnp.where(v_pack >= ma_pack, one, zv)
             * jnp.where(v_pack >= pa, one, zv))

    h0f = jnp.where(hvec == 0, one, zv)
    lane8f = jnp.where(lane == SPECIAL_LANE, one, zv)
    for i in range(G):
        bc = _shuffle(condf, jnp.full((L,), i, jnp.int32))
        ft[i, pl.ds(SPECIAL - FT0, L)] = bc * h0f * lane8f

    writes.append(pltpu.async_copy(
        ft, out_hbm.at[pl.ds(r0, G), pl.ds(c0 + FT0, 128)], semw))

    for w in writes:
        w.wait()


@functools.partial(jax.jit, donate_argnums=())
def _run(inputs):
    mesh = plsc.VectorSubcoreMesh(core_axis_name="c", subcore_axis_name="s")
    f = pl.kernel(
        _body,
        out_type=jax.ShapeDtypeStruct((B, N), jnp.float32),
        mesh=mesh,
        scratch_types=[
            pltpu.VMEM((G, CHUNK), jnp.float32),
            pltpu.VMEM((G, CHUNK), jnp.float32),
            pltpu.VMEM((G, ZW), jnp.float32),
            pltpu.VMEM((G, 128), jnp.float32),
            pltpu.VMEM((L,), jnp.float32),
            pltpu.VMEM_SHARED((NS, L), jnp.float32),
            pltpu.SemaphoreType.DMA,
            pltpu.SemaphoreType.DMA,
            pltpu.SemaphoreType.DMA,
            pltpu.SemaphoreType.DMA,
        ],
    )
    return f(inputs)


def kernel(inputs):
    return _run(inputs)


# speedup vs baseline: 2.1790x; 1.0127x over previous
import functools

import jax
import jax.numpy as jnp
from jax import lax
from jax.experimental import pallas as pl
from jax.experimental.pallas import tpu as pltpu
from jax.experimental.pallas import tpu_sc as plsc

B = 128
N = 32768
KCOL = 1000
L = 16
NEG = float("-inf")

NC = 2
NS = 16
G = 8
NG = B // G
HALF = N // 2

CHUNK = 4096
NCH = HALF // CHUNK

FT0 = (KCOL // 128) * 128
SPECIAL = (KCOL // L) * L
SPECIAL_LANE = KCOL - SPECIAL
SPECIAL_VREG = SPECIAL // L

ZW = 1024


_DNUMS = lax.GatherDimensionNumbers(
    offset_dims=(), collapsed_slice_dims=(0,), start_index_map=(0,))


def _shuffle(x, idx):
    return lax.gather(x, jnp.reshape(idx, (L, 1)), _DNUMS, slice_sizes=(1,),
                      mode=lax.GatherScatterMode.PROMISE_IN_BOUNDS)


def _rows_max(ref, lo, hi, accs):
    if hi <= lo:
        return tuple(accs)
    if hi - lo <= 2:
        a = list(accs)
        for j in range(lo, hi):
            for i in range(G):
                a[i] = jnp.maximum(a[i], ref[i, pl.ds(j * L, L)])
        return tuple(a)

    def body(j, a8):
        base = j * L
        a = list(a8)
        for i in range(G):
            a[i] = jnp.maximum(a[i], ref[i, pl.ds(base, L)])
        return tuple(a)
    return lax.fori_loop(lo, hi, body, tuple(accs))


def _zero_fill(ref, w):
    zv = jnp.zeros((L,), jnp.float32)

    def body(j, c):
        for i in range(G):
            ref[i, pl.ds(j * L, L)] = zv
        return c
    lax.fori_loop(0, w // L, body, 0)


def _body(in_hbm, out_hbm, ibuf0, ibuf1, zb, ft, xbuf, shared,
          sem0, sem1, semw, semx):
    c = lax.axis_index("c")
    s = lax.axis_index("s")
    g = c * (NG // NC) + s // 2
    h = s % 2
    r0 = pl.multiple_of(g * G, G)
    c0 = pl.multiple_of(h * HALF, HALF)

    ibufs = (ibuf0, ibuf1)
    insems = (sem0, sem1)
    lane = lax.iota(jnp.int32, L)
    neg = jnp.full((L,), NEG, jnp.float32)
    zv = jnp.zeros((L,), jnp.float32)
    one = jnp.full((L,), 1.0, jnp.float32)
    half_one = jnp.full((L,), 0.5, jnp.float32)
    hvec = jnp.full((L,), h, jnp.int32)

    _zero_fill(zb, ZW)
    _zero_fill(ft, 128)

    copies = [None] * NCH
    copies[0] = pltpu.async_copy(
        in_hbm.at[pl.ds(r0, G), pl.ds(c0, CHUNK)], ibufs[0], insems[0])

    wplan = [(0, FT0)] + [(k * ZW, ZW) for k in range(1, HALF // ZW)]
    writes = []

    def fire_writes(n):
        while wplan and n > 0:
            off, w = wplan.pop(0)
            src = zb.at[:, pl.ds(0, w)] if w != ZW else zb
            writes.append(pltpu.async_copy(
                src, out_hbm.at[pl.ds(r0, G), pl.ds(c0 + off, w)], semw))
            n -= 1

    acc_b = [neg] * G
    acc_a = [neg] * G
    xs = [None] * G
    for ch in range(NCH):
        if ch + 1 < NCH:
            copies[ch + 1] = pltpu.async_copy(
                in_hbm.at[pl.ds(r0, G), pl.ds(c0 + (ch + 1) * CHUNK, CHUNK)],
                ibufs[(ch + 1) % 2], insems[(ch + 1) % 2])
        fire_writes(4)
        copies[ch].wait()
        ibuf = ibufs[ch % 2]
        if ch == 0:
            acc_b = list(_rows_max(ibuf, 0, SPECIAL_VREG, acc_b))
            acc_a = list(_rows_max(ibuf, SPECIAL_VREG + 1, CHUNK // L, acc_a))
            xs = [ibuf[i, pl.ds(SPECIAL, L)] for i in range(G)]
        else:
            acc_a = list(_rows_max(ibuf, 0, CHUNK // L, acc_a))
    fire_writes(len(wplan))

    keepf = jnp.maximum(jnp.where(lane > SPECIAL_LANE, one, zv),
                        jnp.where(hvec == 1, one, zv))
    for i in range(G):
        x = xs[i]
        acc_b[i] = jnp.maximum(acc_b[i],
                               jnp.where(lane < SPECIAL_LANE, x, neg))
        acc_a[i] = jnp.maximum(acc_a[i], jnp.where(keepf > half_one, x, neg))

    for i in range(G):
        xbuf[i, pl.ds(0, L)] = acc_a[i]
    pltpu.async_copy(xbuf, shared.at[s], semx).wait()
    plsc.subcore_barrier()
    pltpu.async_copy(shared.at[jnp.bitwise_xor(s, 1)], xbuf, semx).wait()
    for i in range(G):
        acc_a[i] = jnp.maximum(acc_a[i], xbuf[i, pl.ds(0, L)])

    h0f = jnp.where(hvec == 0, one, zv)
    lane8f = jnp.where(lane == SPECIAL_LANE, one, zv)
    sp_idx = jnp.full((L,), SPECIAL_LANE, jnp.int32)
    for i in range(G):
        vv = _shuffle(xs[i], sp_idx)
        pf = (jnp.where(vv > acc_b[i], one, zv)
              * jnp.where(vv >= acc_a[i], one, zv))
        for st in (1, 2, 4, 8):
            pf = jnp.minimum(pf, _shuffle(pf, jnp.bitwise_xor(lane, st)))
        ft[i, pl.ds(SPECIAL - FT0, L)] = pf * h0f * lane8f

    writes.append(pltpu.async_copy(
        ft, out_hbm.at[pl.ds(r0, G), pl.ds(c0 + FT0, 128)], semw))

    for w in writes:
        w.wait()


@functools.partial(jax.jit, donate_argnums=())
def _run(inputs):
    mesh = plsc.VectorSubcoreMesh(core_axis_name="c", subcore_axis_name="s")
    f = pl.kernel(
        _body,
        out_type=jax.ShapeDtypeStruct((B, N), jnp.float32),
        mesh=mesh,
        scratch_types=[
            pltpu.VMEM((G, CHUNK), jnp.float32),
            pltpu.VMEM((G, CHUNK), jnp.float32),
            pltpu.VMEM((G, ZW), jnp.float32),
            pltpu.VMEM((G, 128), jnp.float32),
            pltpu.VMEM((G, L), jnp.float32),
            pltpu.VMEM_SHARED((NS, G, L), jnp.float32),
            pltpu.SemaphoreType.DMA,
            pltpu.SemaphoreType.DMA,
            pltpu.SemaphoreType.DMA,
            pltpu.SemaphoreType.DMA,
        ],
    )
    return f(inputs)


def kernel(inputs):
    return _run(inputs)


# speedup vs baseline: 5.3727x; 2.4657x over previous
import functools

import jax
import jax.numpy as jnp
from jax import lax
from jax.experimental import pallas as pl
from jax.experimental.pallas import tpu as pltpu

B = 128
N = 32768
KCOL = 1000
BN = 4096
NB = N // BN
NEG = float("-inf")


def _fold(data):
    m = data[:, 0:128]
    for j in range(1, BN // 128):
        m = jnp.maximum(m, data[:, j * 128:(j + 1) * 128])
    return m


def _body(in_ref, out_ref, acc_ref):
    i = pl.program_id(0)

    @pl.when(i == 0)
    def _():
        acc_ref[...] = jnp.full((B, 128), NEG, jnp.float32)

    data = in_ref[...]

    @pl.when(i < NB - 1)
    def _():
        acc_ref[...] = jnp.maximum(acc_ref[...], _fold(data))
        out_ref[...] = jnp.zeros((B, BN), jnp.float32)

    @pl.when(i == NB - 1)
    def _():
        col = lax.broadcasted_iota(jnp.int32, (B, BN), 1)
        neg = jnp.float32(NEG)
        m_b = jnp.max(jnp.where(col < KCOL, data, neg), axis=1)
        m_a0 = jnp.max(jnp.where(col > KCOL, data, neg), axis=1)
        v = data[:, KCOL]
        m_a = jnp.maximum(m_a0, jnp.max(acc_ref[...], axis=1))
        flag = jnp.where(jnp.logical_and(v > m_b, v >= m_a),
                         jnp.float32(1.0), jnp.float32(0.0))
        out_ref[...] = jnp.where(col == KCOL, flag[:, None],
                                 jnp.float32(0.0))


@jax.jit
def _run(inputs):
    def idx(i):
        return (0, (i + 1) % NB)

    return pl.pallas_call(
        _body,
        grid=(NB,),
        in_specs=[pl.BlockSpec((B, BN), idx)],
        out_specs=pl.BlockSpec((B, BN), idx),
        out_shape=jax.ShapeDtypeStruct((B, N), jnp.float32),
        scratch_shapes=[pltpu.VMEM((B, 128), jnp.float32)],
    )(inputs)


def kernel(inputs):
    return _run(inputs)
